# baseline (speedup 1.0000x reference)
import functools

import jax
import jax.numpy as jnp
from jax import lax
from jax.experimental import pallas as pl
from jax.experimental.pallas import tpu as pltpu
from jax.experimental.pallas import tpu_sc as plsc

_NC = 2
_NS = 16
_LANES = 16


def _make_sc_kernel(n, actions, emb, delta):
    nw = _NC * _NS
    npairs = n * actions
    assert n % nw == 0
    rows_per_w = n // nw
    pairs_per_w = rows_per_w * actions
    crows = 4
    chunk = crows * actions
    assert rows_per_w % (2 * crows) == 0
    assert chunk % _LANES == 0
    nchunks = rows_per_w // crows
    groups_per_chunk = chunk // _LANES
    nrows = (delta + 1) * actions
    row0 = delta * actions

    mesh = plsc.VectorSubcoreMesh(core_axis_name="c", subcore_axis_name="s")

    @functools.partial(
        pl.kernel,
        out_type=jax.ShapeDtypeStruct((n, actions, emb), jnp.float32),
        mesh=mesh,
        compiler_params=pltpu.CompilerParams(
            needs_layout_passes=False, use_tc_tiling_on_sc=False
        ),
        scratch_types=[
            pltpu.VMEM((nrows, emb), jnp.float32),
            pltpu.VMEM((crows, actions), jnp.float32),
            pltpu.VMEM((crows, actions), jnp.float32),
            pltpu.VMEM((chunk, emb), jnp.float32),
            pltpu.VMEM((chunk, emb), jnp.float32),
            pltpu.SemaphoreType.DMA,
            pltpu.SemaphoreType.DMA,
            pltpu.SemaphoreType.DMA,
            pltpu.SemaphoreType.DMA,
        ],
    )
    def sc_kernel(x_in, b_in, out_nd, table_v, xb0, xb1, st0, st1,
                  sem0, sem1, xsem0, xsem1):
        wid = lax.axis_index("s") * _NC + lax.axis_index("c")
        base_row = wid * rows_per_w
        pltpu.sync_copy(b_in.at[pl.ds(row0, nrows)], table_v)

        lanes = lax.iota(jnp.int32, 16)
        scale = jnp.full((16,), float(delta), jnp.float32)
        one = jnp.full((16,), 1.0, jnp.float32)
        av = jnp.full((16,), actions, jnp.int32)

        def xrows(cb):
            b0 = jnp.minimum(base_row + cb // actions, n - crows)
            return x_in.at[pl.ds(b0, crows)]

        def fill_chunk(chunk_base, xbuf, stage_v):
            @pl.loop(0, groups_per_chunk)
            def _group_loop(gi):
                loc = gi * _LANES + lanes
                r = lax.div(loc, av)
                a = loc - r * av
                xv = plsc.load_gather(xbuf, [r, a])
                t = xv * scale
                fl = t.astype(jnp.int32)
                fh = (t + one).astype(jnp.int32)
                wh = t - fl.astype(jnp.float32)
                wl = fh.astype(jnp.float32) - t
                lo_row = fl * av + a
                hi_row = fh * av + a

                @plsc.parallel_loop(0, emb, unroll=8)
                def _col_loop(s):
                    col = lax.bitwise_xor(
                        jnp.full((16,), s, jnp.int32), lanes
                    )
                    lo = plsc.load_gather(table_v, [lo_row, col])
                    hi = plsc.load_gather(table_v, [hi_row, col])
                    o = wl * lo + wh * hi
                    plsc.store_scatter(stage_v, [loc, col], o)

        def row_copies(chunk_base, stage_v, sem, start):
            b0 = base_row + chunk_base // actions
            for k in range(crows):
                src = stage_v.at[pl.ds(k * actions, actions)]
                desc = pltpu.make_async_copy(src, out_nd.at[b0 + k], sem)
                if start:
                    desc.start()
                else:
                    desc.wait()

        def process(cb, xbuf, xsem, stage_v, sem, drain):
            pltpu.make_async_copy(xrows(cb), xbuf, xsem).wait()
            if drain:
                row_copies(cb, stage_v, sem, start=False)
            fill_chunk(cb, xbuf, stage_v)
            row_copies(cb, stage_v, sem, start=True)
            pltpu.async_copy(xrows(cb + 2 * chunk), xbuf, xsem)

        pltpu.async_copy(xrows(0), xb0, xsem0)
        pltpu.async_copy(xrows(chunk), xb1, xsem1)

        process(0, xb0, xsem0, st0, sem0, False)
        process(chunk, xb1, xsem1, st1, sem1, False)

        @pl.loop(2, nchunks, step=2)
        def _chunk_loop(ci):
            process(ci * chunk, xb0, xsem0, st0, sem0, True)
            process((ci + 1) * chunk, xb1, xsem1, st1, sem1, True)

        pltpu.make_async_copy(xrows(0), xb0, xsem0).wait()
        pltpu.make_async_copy(xrows(0), xb1, xsem1).wait()
        row_copies(0, st0, sem0, start=False)
        row_copies(0, st1, sem1, start=False)

    return sc_kernel


def kernel(x, b):
    n, actions = x.shape
    emb = b.shape[1]
    delta = (b.shape[0] // actions - 1) // 2
    sc = _make_sc_kernel(n, actions, emb, delta)
    return sc(x, b)


# speedup vs baseline: 2.5824x; 2.5824x over previous
import functools

import jax
import jax.numpy as jnp
from jax import lax
from jax.experimental import pallas as pl
from jax.experimental.pallas import tpu as pltpu
from jax.experimental.pallas import tpu_sc as plsc

_NC = 2
_NS = 16
_LANES = 16


def _make_sc_kernel(n, actions, emb, delta):
    nw = _NC * _NS
    nf = delta + 1
    nfp = _LANES
    et_n = emb // 8
    units = actions * et_n
    assert units % nw == 0
    upw = units // nw
    ntiles = n // 128
    assert n % 128 == 0
    nbv = n // _LANES
    na = (upw + et_n - 2) // et_n + 1

    mesh = plsc.VectorSubcoreMesh(core_axis_name="c", subcore_axis_name="s")

    @functools.partial(
        pl.kernel,
        out_type=jax.ShapeDtypeStruct((actions, et_n, ntiles, 8, 128),
                                      jnp.float32),
        mesh=mesh,
        compiler_params=pltpu.CompilerParams(
            needs_layout_passes=False, use_tc_tiling_on_sc=False
        ),
        scratch_types=[
            pltpu.VMEM((emb, na, nfp), jnp.float32),
            pltpu.VMEM((emb * na * nf * _LANES,), jnp.float32),
            pltpu.VMEM((n,), jnp.float32),
            pltpu.VMEM((n,), jnp.float32),
            pltpu.VMEM((ntiles, 8, 128), jnp.float32),
            pltpu.VMEM((ntiles, 8, 128), jnp.float32),
            pltpu.SemaphoreType.DMA,
            pltpu.SemaphoreType.DMA,
            pltpu.SemaphoreType.DMA,
            pltpu.SemaphoreType.DMA,
        ],
    )
    def sc_kernel(xt_in, btp_in, out5, tmp_v, table_v, xb0, xb1, st0, st1,
                  sem0, sem1, xsem0, xsem1):
        wid = lax.axis_index("s") * _NC + lax.axis_index("c")
        u0 = wid * upw
        a0 = jnp.minimum(u0 // et_n, actions - na)

        pltpu.sync_copy(btp_in.at[:, pl.ds(a0, na)], tmp_v)

        lanes = lax.iota(jnp.int32, 16)
        scale = jnp.full((16,), float(delta), jnp.float32)
        one = jnp.full((16,), 1.0, jnp.float32)

        @pl.loop(0, emb)
        def _expand(e):
            for al in range(na):
                v = tmp_v[e, al]
                base = ((e * na + al) * nf) * _LANES
                for f in range(nf):
                    bc = jnp.take(v, jnp.full((16,), f, jnp.int32))
                    table_v[pl.ds(base + f * _LANES, _LANES)] = bc

        def unit_of(h):
            u = u0 + h
            return u // et_n, u % et_n

        def xrow(h):
            a, _ = unit_of(jnp.minimum(h, upw - 1))
            return xt_in.at[a]

        def fill(h, xbuf, stage_v):
            a, et = unit_of(h)
            al = a - a0
            e0 = et * 8

            @plsc.parallel_loop(0, nbv, unroll=2)
            def _batch_loop(nb):
                xv = xbuf[pl.ds(nb * _LANES, _LANES)]
                t = xv * scale
                fl = t.astype(jnp.int32)
                fh = (t + one).astype(jnp.int32)
                wl = fh.astype(jnp.float32) - t
                wh = t - fl.astype(jnp.float32)
                flo = fl * _LANES + lanes
                fhi = fh * _LANES + lanes
                snt = nb // 8
                soff = (nb % 8) * _LANES
                los, his = [], []
                for ei in range(8):
                    base = jnp.full((16,), 0, jnp.int32) + (
                        ((e0 + ei) * na + al) * (nf * _LANES)
                    )
                    los.append(plsc.load_gather(table_v, [base + flo]))
                    his.append(plsc.load_gather(table_v, [base + fhi]))
                outs = [wl * lo + wh * hi for lo, hi in zip(los, his)]
                for ei in range(8):
                    stage_v[snt, ei, pl.ds(soff, _LANES)] = outs[ei]

        def out_slice(h):
            a, et = unit_of(h)
            return out5.at[a, et]

        def process(h, xbuf, xsem, stage_v, sem, drain):
            pltpu.make_async_copy(xrow(h), xbuf, xsem).wait()
            if drain:
                pltpu.make_async_copy(stage_v, out_slice(h), sem).wait()
            fill(h, xbuf, stage_v)
            pltpu.async_copy(stage_v, out_slice(h), sem)
            pltpu.async_copy(xrow(h + 2), xbuf, xsem)

        pltpu.async_copy(xrow(0), xb0, xsem0)
        pltpu.async_copy(xrow(1), xb1, xsem1)

        process(0, xb0, xsem0, st0, sem0, False)
        process(1, xb1, xsem1, st1, sem1, False)

        @pl.loop(2, upw - 1, step=2)
        def _unit_loop(h):
            process(h, xb0, xsem0, st0, sem0, True)
            process(h + 1, xb1, xsem1, st1, sem1, True)

        process(upw - 1, xb0, xsem0, st0, sem0, True)

        pltpu.make_async_copy(xrow(0), xb0, xsem0).wait()
        pltpu.make_async_copy(xrow(0), xb1, xsem1).wait()
        pltpu.make_async_copy(st0, out_slice(0), sem0).wait()
        pltpu.make_async_copy(st1, out_slice(0), sem1).wait()

    return sc_kernel


def kernel(x, b):
    n, actions = x.shape
    emb = b.shape[1]
    delta = (b.shape[0] // actions - 1) // 2
    nf = delta + 1
    xt = x.T
    btp = jnp.pad(
        b[delta * actions:(2 * delta + 1) * actions]
        .reshape(nf, actions, emb)
        .transpose(2, 1, 0),
        ((0, 0), (0, 0), (0, 16 - nf)),
    )
    sc = _make_sc_kernel(n, actions, emb, delta)
    h5 = sc(xt, btp)
    return h5.transpose(2, 4, 0, 1, 3).reshape(n, actions, emb)
